# baseline (speedup 1.0000x reference)
import functools

import jax
import jax.numpy as jnp
from jax import lax
from jax.experimental import pallas as pl
from jax.experimental.pallas import tpu as pltpu
from jax.experimental.pallas import tpu_sc as plsc

C = 64
NBINS = C + 1
ALPHA_LO = 0.5 - 0.1
ALPHA_HI = 0.5 + 0.1
NC = 2
NS = 16
NW = NC * NS
L = 16


def _sc_body(B, E, R, x_hbm, lr_hbm, out_hbm,
             xt0, xt1, lrt, hist, ot, sem0, sem1):
    rows_per_w = B // NW
    nblocks = rows_per_w // R
    nchunks = E // L

    cid = lax.axis_index("c")
    sid = lax.axis_index("s")
    wid = sid * NC + cid
    row0 = wid * rows_per_w

    lane = lax.iota(jnp.int32, L)
    lane_rot = lane * (E + 1)
    lane_out = lane * C
    ones = jnp.ones((L,), jnp.int32)
    zeros = jnp.zeros((L,), jnp.int32)
    center16 = jnp.full((L,), C * L, jnp.int32)
    xwin = (L - 1) * (E + 1) + L

    pltpu.sync_copy(lr_hbm, lrt)

    for i in range((R * C) // L):
        ot[pl.ds(i * L, L)] = zeros

    def x_rows(b):
        return x_hbm.at[pl.ds((row0 + b * R) * E, R * E)]

    def hist_step(xv, lrp):
        rb = lax.shift_right_logical(lrp, 16)
        lb = lrp & 0xFFFF
        below = xv <= ALPHA_LO
        above = xv >= ALPHA_HI
        sel16 = jnp.where(below, lb, jnp.where(above, rb, center16))
        plsc.addupdate_scatter(hist, [sel16 + lane], ones)

    def compute_block(bi, xt):
        for i in range((L * NBINS) // L):
            hist[pl.ds(i * L, L)] = zeros

        def per_chunk(ch):
            ch16 = ch * L
            xs = xt.at[pl.ds(ch16, xwin)]
            lridx0 = lane + ch16
            for u in range(L):
                lrp = plsc.load_gather(lrt, [lridx0 + u])
                xv = plsc.load_gather(xs, [lane_rot + u])
                hist_step(xv, lrp)
        plsc.parallel_loop(0, nchunks - 1, unroll=5)(per_chunk)

        for u in range(L):
            e = E - L + u
            lrp = plsc.load_gather(lrt, [lane + e])
            wrap = (lane + e) >= E
            xidx = lane_rot + e - jnp.where(wrap, E, 0)
            xv = plsc.load_gather(xt, [xidx])
            hist_step(xv, lrp)

        m = jnp.full((L,), -1, jnp.int32)
        am = zeros
        for c in range(C):
            v = hist[pl.ds(c * L, L)]
            better = v > m
            m = jnp.where(better, v, m)
            am = jnp.where(better, jnp.full((L,), c * L, jnp.int32), am)
        am = lax.shift_right_logical(am, 4)

        plsc.store_scatter(ot, [lane_out + am], ones)
        pltpu.sync_copy(ot, out_hbm.at[pl.ds((row0 + bi * R) * C, R * C)])
        plsc.store_scatter(ot, [lane_out + am], zeros)

    pltpu.async_copy(x_rows(0), xt0, sem0)

    def per_pair(g, _):
        b0 = g * 2
        pltpu.async_copy(x_rows(b0 + 1), xt1, sem1)
        pltpu.make_async_copy(x_rows(b0), xt0, sem0).wait()
        compute_block(b0, xt0)
        nxt = jnp.minimum(b0 + 2, nblocks - 1)
        pltpu.async_copy(x_rows(nxt), xt0, sem0)
        pltpu.make_async_copy(x_rows(b0 + 1), xt1, sem1).wait()
        compute_block(b0 + 1, xt1)
        return 0

    lax.fori_loop(0, nblocks // 2, per_pair, 0)
    pltpu.make_async_copy(x_rows(0), xt0, sem0).wait()


def kernel(x, perms):
    B, E = x.shape
    left16 = perms[:, 0].astype(jnp.int32) * L
    right16 = perms[:, 1].astype(jnp.int32) * L
    lrp = left16 | (right16 << 16)
    lrp = jnp.concatenate([lrp, lrp[:L]])
    R = 16

    mesh = plsc.VectorSubcoreMesh(
        core_axis_name="c", subcore_axis_name="s",
        num_cores=NC, num_subcores=NS)

    run = pl.kernel(
        functools.partial(_sc_body, B, E, R),
        out_type=jax.ShapeDtypeStruct((B * C,), jnp.int32),
        mesh=mesh,
        compiler_params=pltpu.CompilerParams(needs_layout_passes=False),
        scratch_types=[
            pltpu.VMEM((R * E,), jnp.float32),
            pltpu.VMEM((R * E,), jnp.float32),
            pltpu.VMEM((E + L,), jnp.int32),
            pltpu.VMEM((NBINS * L,), jnp.int32),
            pltpu.VMEM((R * C,), jnp.int32),
            pltpu.SemaphoreType.DMA,
            pltpu.SemaphoreType.DMA,
        ],
    )
    out = run(x.reshape(B * E), lrp)
    return out.reshape(B, C).astype(jnp.int64)


# speedup vs baseline: 1.0041x; 1.0041x over previous
import functools

import jax
import jax.numpy as jnp
from jax import lax
from jax.experimental import pallas as pl
from jax.experimental.pallas import tpu as pltpu
from jax.experimental.pallas import tpu_sc as plsc

C = 64
NBINS = C + 1
ALPHA_LO = 0.5 - 0.1
ALPHA_HI = 0.5 + 0.1
NC = 2
NS = 16
NW = NC * NS
L = 16


def _sc_body(B, E, R, x_hbm, lr_hbm, out_hbm,
             xt0, xt1, lrt, hist, ot, sem0, sem1):
    rows_per_w = B // NW
    nblocks = rows_per_w // R
    nchunks = E // L

    cid = lax.axis_index("c")
    sid = lax.axis_index("s")
    wid = sid * NC + cid
    row0 = wid * rows_per_w

    lane = lax.iota(jnp.int32, L)
    lane_rot = lane * (E + 1)
    lane_out = lane * C
    ones = jnp.ones((L,), jnp.int32)
    zeros = jnp.zeros((L,), jnp.int32)
    center16 = jnp.full((L,), C * L, jnp.int32)
    xwin = (L - 1) * (E + 1) + L

    pltpu.sync_copy(lr_hbm, lrt)

    for i in range((R * C) // L):
        ot[pl.ds(i * L, L)] = zeros

    def x_rows(b):
        return x_hbm.at[pl.ds((row0 + b * R) * E, R * E)]

    def hist_step(xv, lrp):
        rb = lax.shift_right_logical(lrp, 16)
        lb = lrp & 0xFFFF
        below = xv <= ALPHA_LO
        above = xv >= ALPHA_HI
        sel16 = jnp.where(below, lb, jnp.where(above, rb, center16))
        plsc.addupdate_scatter(hist, [sel16 + lane], ones)

    def compute_block(bi, xt):
        for i in range((L * NBINS) // L):
            hist[pl.ds(i * L, L)] = zeros

        def per_chunk(ch):
            ch16 = ch * L
            xs = xt.at[pl.ds(ch16, xwin)]
            lridx0 = lane + ch16
            for u in range(L):
                lrp = plsc.load_gather(lrt, [lridx0 + u])
                xv = plsc.load_gather(xs, [lane_rot + u])
                hist_step(xv, lrp)
        plsc.parallel_loop(0, nchunks - 1, unroll=2)(per_chunk)

        for u in range(L):
            e = E - L + u
            lrp = plsc.load_gather(lrt, [lane + e])
            wrap = (lane + e) >= E
            xidx = lane_rot + e - jnp.where(wrap, E, 0)
            xv = plsc.load_gather(xt, [xidx])
            hist_step(xv, lrp)

        m = jnp.full((L,), -1, jnp.int32)
        am = zeros
        for c in range(C):
            v = hist[pl.ds(c * L, L)]
            better = v > m
            m = jnp.where(better, v, m)
            am = jnp.where(better, jnp.full((L,), c * L, jnp.int32), am)
        am = lax.shift_right_logical(am, 4)

        plsc.store_scatter(ot, [lane_out + am], ones)
        pltpu.sync_copy(ot, out_hbm.at[pl.ds((row0 + bi * R) * C, R * C)])
        plsc.store_scatter(ot, [lane_out + am], zeros)

    pltpu.async_copy(x_rows(0), xt0, sem0)

    def per_pair(g, _):
        b0 = g * 2
        pltpu.async_copy(x_rows(b0 + 1), xt1, sem1)
        pltpu.make_async_copy(x_rows(b0), xt0, sem0).wait()
        compute_block(b0, xt0)
        nxt = jnp.minimum(b0 + 2, nblocks - 1)
        pltpu.async_copy(x_rows(nxt), xt0, sem0)
        pltpu.make_async_copy(x_rows(b0 + 1), xt1, sem1).wait()
        compute_block(b0 + 1, xt1)
        return 0

    lax.fori_loop(0, nblocks // 2, per_pair, 0)
    pltpu.make_async_copy(x_rows(0), xt0, sem0).wait()


def kernel(x, perms):
    B, E = x.shape
    left16 = perms[:, 0].astype(jnp.int32) * L
    right16 = perms[:, 1].astype(jnp.int32) * L
    lrp = left16 | (right16 << 16)
    lrp = jnp.concatenate([lrp, lrp[:L]])
    R = 16

    mesh = plsc.VectorSubcoreMesh(
        core_axis_name="c", subcore_axis_name="s",
        num_cores=NC, num_subcores=NS)

    run = pl.kernel(
        functools.partial(_sc_body, B, E, R),
        out_type=jax.ShapeDtypeStruct((B * C,), jnp.int32),
        mesh=mesh,
        compiler_params=pltpu.CompilerParams(needs_layout_passes=False),
        scratch_types=[
            pltpu.VMEM((R * E,), jnp.float32),
            pltpu.VMEM((R * E,), jnp.float32),
            pltpu.VMEM((E + L,), jnp.int32),
            pltpu.VMEM((NBINS * L,), jnp.int32),
            pltpu.VMEM((R * C,), jnp.int32),
            pltpu.SemaphoreType.DMA,
            pltpu.SemaphoreType.DMA,
        ],
    )
    out = run(x.reshape(B * E), lrp)
    return out.reshape(B, C).astype(jnp.int64)


# speedup vs baseline: 1.0412x; 1.0369x over previous
import functools

import jax
import jax.numpy as jnp
from jax import lax
from jax.experimental import pallas as pl
from jax.experimental.pallas import tpu as pltpu
from jax.experimental.pallas import tpu_sc as plsc

C = 64
NBINS = C + 1
ALPHA_LO = 0.5 - 0.1
ALPHA_HI = 0.5 + 0.1
NC = 2
NS = 16
NW = NC * NS
L = 16


def _sc_body(B, E, R, x_hbm, lr_hbm, out_hbm,
             xt0, xt1, lrt, hist, ot, sem0, sem1):
    rows_per_w = B // NW
    nblocks = rows_per_w // R
    nchunks = E // L

    cid = lax.axis_index("c")
    sid = lax.axis_index("s")
    wid = sid * NC + cid
    row0 = wid * rows_per_w

    lane = lax.iota(jnp.int32, L)
    lane_rot = lane * (E + 1)
    lane_out = lane * C
    ones = jnp.ones((L,), jnp.int32)
    zeros = jnp.zeros((L,), jnp.int32)
    xwin = (L - 1) * (E + 1) + L

    pltpu.sync_copy(lr_hbm, lrt)

    for i in range((R * C) // L):
        ot[pl.ds(i * L, L)] = zeros

    def x_rows(b):
        return x_hbm.at[pl.ds((row0 + b * R) * E, R * E)]

    def hist_step(xv, lrp):
        rbl = lax.shift_right_logical(lrp, 16) + lane
        lbl = (lrp & 0xFFFF) + lane
        below = xv <= ALPHA_LO
        above = xv >= ALPHA_HI
        plsc.addupdate_scatter(hist, [lbl], ones, mask=below)
        plsc.addupdate_scatter(hist, [rbl], ones, mask=above)

    def compute_block(bi, xt):
        for i in range(C):
            hist[pl.ds(i * L, L)] = zeros

        def per_chunk(ch):
            ch16 = ch * L
            xs = xt.at[pl.ds(ch16, xwin)]
            lridx0 = lane + ch16
            for u in range(L):
                lrp = plsc.load_gather(lrt, [lridx0 + u])
                xv = plsc.load_gather(xs, [lane_rot + u])
                hist_step(xv, lrp)
        plsc.parallel_loop(0, nchunks - 1)(per_chunk)

        for u in range(L):
            e = E - L + u
            lrp = plsc.load_gather(lrt, [lane + e])
            wrap = (lane + e) >= E
            xidx = lane_rot + e - jnp.where(wrap, E, 0)
            xv = plsc.load_gather(xt, [xidx])
            hist_step(xv, lrp)

        m = jnp.full((L,), -1, jnp.int32)
        am = zeros
        for c in range(C):
            v = hist[pl.ds(c * L, L)]
            better = v > m
            m = jnp.where(better, v, m)
            am = jnp.where(better, jnp.full((L,), c * L, jnp.int32), am)
        am = lax.shift_right_logical(am, 4)

        plsc.store_scatter(ot, [lane_out + am], ones)
        pltpu.sync_copy(ot, out_hbm.at[pl.ds((row0 + bi * R) * C, R * C)])
        plsc.store_scatter(ot, [lane_out + am], zeros)

    pltpu.async_copy(x_rows(0), xt0, sem0)

    def per_pair(g, _):
        b0 = g * 2
        pltpu.async_copy(x_rows(b0 + 1), xt1, sem1)
        pltpu.make_async_copy(x_rows(b0), xt0, sem0).wait()
        compute_block(b0, xt0)
        nxt = jnp.minimum(b0 + 2, nblocks - 1)
        pltpu.async_copy(x_rows(nxt), xt0, sem0)
        pltpu.make_async_copy(x_rows(b0 + 1), xt1, sem1).wait()
        compute_block(b0 + 1, xt1)
        return 0

    lax.fori_loop(0, nblocks // 2, per_pair, 0)
    pltpu.make_async_copy(x_rows(0), xt0, sem0).wait()


def kernel(x, perms):
    B, E = x.shape
    left16 = perms[:, 0].astype(jnp.int32) * L
    right16 = perms[:, 1].astype(jnp.int32) * L
    lrp = left16 | (right16 << 16)
    lrp = jnp.concatenate([lrp, lrp[:L]])
    R = 16

    mesh = plsc.VectorSubcoreMesh(
        core_axis_name="c", subcore_axis_name="s",
        num_cores=NC, num_subcores=NS)

    run = pl.kernel(
        functools.partial(_sc_body, B, E, R),
        out_type=jax.ShapeDtypeStruct((B * C,), jnp.int32),
        mesh=mesh,
        compiler_params=pltpu.CompilerParams(needs_layout_passes=False),
        scratch_types=[
            pltpu.VMEM((R * E,), jnp.float32),
            pltpu.VMEM((R * E,), jnp.float32),
            pltpu.VMEM((E + L,), jnp.int32),
            pltpu.VMEM((C * L,), jnp.int32),
            pltpu.VMEM((R * C,), jnp.int32),
            pltpu.SemaphoreType.DMA,
            pltpu.SemaphoreType.DMA,
        ],
    )
    out = run(x.reshape(B * E), lrp)
    return out.reshape(B, C).astype(jnp.int64)


# speedup vs baseline: 1.1251x; 1.0806x over previous
import functools

import jax
import jax.numpy as jnp
from jax import lax
from jax.experimental import pallas as pl
from jax.experimental.pallas import tpu as pltpu
from jax.experimental.pallas import tpu_sc as plsc

C = 64
NBINS = C + 1
ALPHA_LO = 0.5 - 0.1
ALPHA_HI = 0.5 + 0.1
NC = 2
NS = 16
NW = NC * NS
L = 16


def _sc_body(B, E, R, x_hbm, lr_hbm, out_hbm,
             xt0, xt1, lrt, hist, ot, sem0, sem1):
    rows_per_w = B // NW
    nblocks = rows_per_w // R
    nchunks = E // L

    cid = lax.axis_index("c")
    sid = lax.axis_index("s")
    wid = sid * NC + cid
    row0 = wid * rows_per_w

    lane = lax.iota(jnp.int32, L)
    lane_rot = lane * (E + 1)
    lane_out = lane * C
    ones = jnp.ones((L,), jnp.int32)
    zeros = jnp.zeros((L,), jnp.int32)
    xwin = (L - 1) * (E + 1) + L

    pltpu.sync_copy(lr_hbm, lrt)

    for i in range((R * C) // L):
        ot[pl.ds(i * L, L)] = zeros

    def x_rows(b):
        return x_hbm.at[pl.ds((row0 + b * R) * E, R * E)]

    center16 = jnp.full((L,), C * L, jnp.int32)

    def hist_step(xv, lrp):
        rb = lax.shift_right_logical(lrp, 16)
        lb = lrp & 0xFFFF
        below = xv <= ALPHA_LO
        above = xv >= ALPHA_HI
        sel16 = jnp.where(below, lb, jnp.where(above, rb, center16))
        plsc.addupdate_scatter(hist, [sel16 + lane], ones)

    def compute_block(bi, xt):
        for i in range(NBINS):
            hist[pl.ds(i * L, L)] = zeros

        def per_elem(e):
            lrp = plsc.load_gather(lrt, [lane + e])
            xv = plsc.load_gather(xt, [lane_rot + e])
            hist_step(xv, lrp)
        plsc.parallel_loop(0, E - L, unroll=8)(per_elem)

        for u in range(L):
            e = E - L + u
            lrp = plsc.load_gather(lrt, [lane + e])
            wrap = (lane + e) >= E
            xidx = lane_rot + e - jnp.where(wrap, E, 0)
            xv = plsc.load_gather(xt, [xidx])
            hist_step(xv, lrp)

        m = jnp.full((L,), -1, jnp.int32)
        am = zeros
        for c in range(C):
            v = hist[pl.ds(c * L, L)]
            better = v > m
            m = jnp.where(better, v, m)
            am = jnp.where(better, jnp.full((L,), c * L, jnp.int32), am)
        am = lax.shift_right_logical(am, 4)

        plsc.store_scatter(ot, [lane_out + am], ones)
        pltpu.sync_copy(ot, out_hbm.at[pl.ds((row0 + bi * R) * C, R * C)])
        plsc.store_scatter(ot, [lane_out + am], zeros)

    pltpu.async_copy(x_rows(0), xt0, sem0)

    def per_pair(g, _):
        b0 = g * 2
        pltpu.async_copy(x_rows(b0 + 1), xt1, sem1)
        pltpu.make_async_copy(x_rows(b0), xt0, sem0).wait()
        compute_block(b0, xt0)
        nxt = jnp.minimum(b0 + 2, nblocks - 1)
        pltpu.async_copy(x_rows(nxt), xt0, sem0)
        pltpu.make_async_copy(x_rows(b0 + 1), xt1, sem1).wait()
        compute_block(b0 + 1, xt1)
        return 0

    lax.fori_loop(0, nblocks // 2, per_pair, 0)
    pltpu.make_async_copy(x_rows(0), xt0, sem0).wait()


def kernel(x, perms):
    B, E = x.shape
    left16 = perms[:, 0].astype(jnp.int32) * L
    right16 = perms[:, 1].astype(jnp.int32) * L
    lrp = left16 | (right16 << 16)
    lrp = jnp.concatenate([lrp, lrp[:L]])
    R = 16

    mesh = plsc.VectorSubcoreMesh(
        core_axis_name="c", subcore_axis_name="s",
        num_cores=NC, num_subcores=NS)

    run = pl.kernel(
        functools.partial(_sc_body, B, E, R),
        out_type=jax.ShapeDtypeStruct((B * C,), jnp.int32),
        mesh=mesh,
        compiler_params=pltpu.CompilerParams(needs_layout_passes=False),
        scratch_types=[
            pltpu.VMEM((R * E,), jnp.float32),
            pltpu.VMEM((R * E,), jnp.float32),
            pltpu.VMEM((E + L,), jnp.int32),
            pltpu.VMEM((NBINS * L,), jnp.int32),
            pltpu.VMEM((R * C,), jnp.int32),
            pltpu.SemaphoreType.DMA,
            pltpu.SemaphoreType.DMA,
        ],
    )
    out = run(x.reshape(B * E), lrp)
    return out.reshape(B, C).astype(jnp.int64)


# speedup vs baseline: 2.4479x; 2.1758x over previous
import functools

import jax
import jax.numpy as jnp
from jax import lax
from jax.experimental import pallas as pl
from jax.experimental.pallas import tpu as pltpu

C = 64
ALPHA_LO = 0.5 - 0.1
ALPHA_HI = 0.5 + 0.1


def _tc_body(xref, lref, rref, oref):
    x = xref[...]
    below = (x <= ALPHA_LO).astype(jnp.bfloat16)
    above = (x >= ALPHA_HI).astype(jnp.bfloat16)
    counts = jnp.dot(below, lref[...], preferred_element_type=jnp.float32)
    counts = counts + jnp.dot(above, rref[...],
                              preferred_element_type=jnp.float32)
    ci = counts.astype(jnp.int32)
    score = ci * C + (C - 1 - lax.broadcasted_iota(jnp.int32, ci.shape, 1))
    mx = jnp.max(score, axis=1, keepdims=True)
    oref[...] = (score == mx).astype(jnp.int32)


def kernel(x, perms):
    B, E = x.shape
    cls = jnp.arange(C, dtype=jnp.int32)
    lmat = (perms[:, 0:1] == cls[None, :]).astype(jnp.bfloat16)
    rmat = (perms[:, 1:2] == cls[None, :]).astype(jnp.bfloat16)
    BT = 512

    out = pl.pallas_call(
        _tc_body,
        grid=(B // BT,),
        in_specs=[
            pl.BlockSpec((BT, E), lambda i: (i, 0)),
            pl.BlockSpec((E, C), lambda i: (0, 0)),
            pl.BlockSpec((E, C), lambda i: (0, 0)),
        ],
        out_specs=pl.BlockSpec((BT, C), lambda i: (i, 0)),
        out_shape=jax.ShapeDtypeStruct((B, C), jnp.int32),
    )(x, lmat, rmat)
    return out.astype(jnp.int64)


# speedup vs baseline: 2.5308x; 1.0339x over previous
import functools

import jax
import jax.numpy as jnp
from jax import lax
from jax.experimental import pallas as pl
from jax.experimental.pallas import tpu as pltpu

C = 64
ALPHA_LO = 0.5 - 0.1
ALPHA_HI = 0.5 + 0.1


def _tc_body(xref, lref, rref, oref):
    x = xref[...]
    below = (x <= ALPHA_LO).astype(jnp.bfloat16)
    above = (x >= ALPHA_HI).astype(jnp.bfloat16)
    counts = jnp.dot(below, lref[...], preferred_element_type=jnp.float32)
    counts = counts + jnp.dot(above, rref[...],
                              preferred_element_type=jnp.float32)
    ci = counts.astype(jnp.int32)
    score = ci * C + (C - 1 - lax.broadcasted_iota(jnp.int32, ci.shape, 1))
    mx = jnp.max(score, axis=1, keepdims=True)
    oref[...] = (score == mx).astype(jnp.int32)


def kernel(x, perms):
    B, E = x.shape
    cls = jnp.arange(C, dtype=jnp.int32)
    lmat = (perms[:, 0:1] == cls[None, :]).astype(jnp.bfloat16)
    rmat = (perms[:, 1:2] == cls[None, :]).astype(jnp.bfloat16)
    BT = 1024

    out = pl.pallas_call(
        _tc_body,
        grid=(B // BT,),
        in_specs=[
            pl.BlockSpec((BT, E), lambda i: (i, 0)),
            pl.BlockSpec((E, C), lambda i: (0, 0)),
            pl.BlockSpec((E, C), lambda i: (0, 0)),
        ],
        out_specs=pl.BlockSpec((BT, C), lambda i: (i, 0)),
        out_shape=jax.ShapeDtypeStruct((B, C), jnp.int32),
    )(x, lmat, rmat)
    return out.astype(jnp.int64)


# speedup vs baseline: 2.6408x; 1.0435x over previous
import functools

import jax
import jax.numpy as jnp
from jax import lax
from jax.experimental import pallas as pl
from jax.experimental.pallas import tpu as pltpu

C = 64
ALPHA_LO = 0.5 - 0.1
ALPHA_HI = 0.5 + 0.1


def _tc_body(xref, lref, rref, oref):
    x = xref[...]
    below = (x <= ALPHA_LO).astype(jnp.bfloat16)
    above = (x >= ALPHA_HI).astype(jnp.bfloat16)
    counts = jnp.dot(below, lref[...], preferred_element_type=jnp.float32)
    counts = counts + jnp.dot(above, rref[...],
                              preferred_element_type=jnp.float32)
    ci = counts.astype(jnp.int32)
    score = ci * C + (C - 1 - lax.broadcasted_iota(jnp.int32, ci.shape, 1))
    mx = jnp.max(score, axis=1, keepdims=True)
    oref[...] = (score == mx).astype(jnp.int32)


def kernel(x, perms):
    B, E = x.shape
    cls = jnp.arange(C, dtype=jnp.int32)
    lmat = (perms[:, 0:1] == cls[None, :]).astype(jnp.bfloat16)
    rmat = (perms[:, 1:2] == cls[None, :]).astype(jnp.bfloat16)
    BT = 2048

    out = pl.pallas_call(
        _tc_body,
        grid=(B // BT,),
        in_specs=[
            pl.BlockSpec((BT, E), lambda i: (i, 0)),
            pl.BlockSpec((E, C), lambda i: (0, 0)),
            pl.BlockSpec((E, C), lambda i: (0, 0)),
        ],
        out_specs=pl.BlockSpec((BT, C), lambda i: (i, 0)),
        out_shape=jax.ShapeDtypeStruct((B, C), jnp.int32),
    )(x, lmat, rmat)
    return out.astype(jnp.int64)
